# baseline (speedup 1.0000x reference)
import functools

import jax
import jax.numpy as jnp
from jax.experimental import pallas as pl
from jax.experimental.pallas import tpu as pltpu
from jax.experimental.pallas import tpu_sc as plsc

_BLK = 1024
_CH = 8
_GW = 256
_EP = 128


def _sc_gather(emb, idx2d):
    n = idx2d.shape[1]
    e = emb.shape[1]
    mesh = plsc.VectorSubcoreMesh(core_axis_name="core", subcore_axis_name="subcore")

    @functools.partial(
        pl.kernel,
        out_type=jax.ShapeDtypeStruct((n, e), emb.dtype),
        mesh=mesh,
    )
    def k(emb_hbm, idx_hbm, out_hbm):
        def body(i_vmem, o_vmem):
            pltpu.sync_copy(emb_hbm.at[i_vmem.at[0]], o_vmem)

        pltpu.emit_pipeline(
            body,
            grid=(n // _GW,),
            in_specs=[pl.BlockSpec((1, _GW), index_map=lambda i: (0, i))],
            out_specs=[pl.BlockSpec((_GW, e), index_map=lambda i: (i, 0))],
            core_axis_name=("core", "subcore"),
            dimension_semantics=(pltpu.PARALLEL,),
        )(idx_hbm, out_hbm)

    return k(emb, idx2d)


def _gru_pallas(xe3, ls_col, lastchunk, wihT, whhT, b2):
    Bs, L, E = xe3.shape
    H = whhT.shape[0]
    G = whhT.shape[1]
    R = Bs // _BLK
    NT = L // _CH

    def body(s_ref, xe_ref, len_ref, wih_ref, whh_ref, b_ref, o_ref, h_ref):
        r = pl.program_id(0)
        c = pl.program_id(1)

        @pl.when(c == 0)
        def _init():
            h_ref[...] = jnp.zeros_like(h_ref)

        @pl.when(c <= s_ref[r])
        def _compute():
            h = h_ref[...]
            lens = len_ref[...]
            bih = b_ref[0:1, :]
            bhh = b_ref[1:2, :]
            for tt in range(_CH):
                t = c * _CH + tt
                xe_t = xe_ref[:, tt, :]
                gi = jnp.dot(xe_t, wih_ref[...],
                             preferred_element_type=jnp.float32) + bih
                gh = jnp.dot(h, whh_ref[...],
                             preferred_element_type=jnp.float32) + bhh
                gsum = gi + gh
                rz = jax.nn.sigmoid(gsum[:, : 2 * H])
                rr = rz[:, :H]
                zz = rz[:, H:]
                n = jnp.tanh(gi[:, 2 * H:] + rr * gh[:, 2 * H:])
                h_new = (1.0 - zz) * n + zz * h
                h = jnp.where(lens > t, h_new, h)
            h_ref[...] = h

        o_ref[...] = h_ref[...]

    return pl.pallas_call(
        body,
        grid_spec=pltpu.PrefetchScalarGridSpec(
            num_scalar_prefetch=1,
            grid=(R, NT),
            in_specs=[
                pl.BlockSpec((_BLK, _CH, E),
                             lambda r, c, s: (r, jnp.minimum(c, s[r]), 0)),
                pl.BlockSpec((_BLK, 1), lambda r, c, s: (r, 0)),
                pl.BlockSpec((E, G), lambda r, c, s: (0, 0)),
                pl.BlockSpec((H, G), lambda r, c, s: (0, 0)),
                pl.BlockSpec((8, G), lambda r, c, s: (0, 0)),
            ],
            out_specs=pl.BlockSpec((_BLK, H), lambda r, c, s: (r, 0)),
            scratch_shapes=[pltpu.VMEM((_BLK, H), jnp.float32)],
        ),
        out_shape=jax.ShapeDtypeStruct((Bs, H), jnp.float32),
        compiler_params=pltpu.CompilerParams(
            dimension_semantics=("parallel", "arbitrary")),
    )(lastchunk, xe3, ls_col, wihT, whhT, b2)


def kernel(x, emb, W_ih, W_hh, b_ih, b_hh):
    B, L = x.shape
    V, E = emb.shape
    H = W_hh.shape[1]

    l = jnp.sum(x != 0, axis=1).astype(jnp.int32)
    perm = jnp.argsort(-l)
    ls = l[perm]
    xs = x[perm]

    emb_p = jnp.pad(emb, ((0, 0), (0, _EP - E)))
    xe = _sc_gather(emb_p, xs.reshape(1, B * L))
    xe3 = xe.reshape(B, L, _EP)

    block_max = ls.reshape(B // _BLK, _BLK).max(axis=1)
    lastchunk = (jnp.maximum((block_max + _CH - 1) // _CH, 1) - 1).astype(jnp.int32)
    ls_col = ls[:, None]
    b2 = jnp.zeros((8, 3 * H), jnp.float32).at[0].set(b_ih).at[1].set(b_hh)

    wihT_p = jnp.pad(W_ih.T, ((0, _EP - E), (0, 0)))
    h = _gru_pallas(xe3, ls_col, lastchunk, wihT_p, W_hh.T, b2)

    nonempty = l != 0
    dest = jnp.where(nonempty, jnp.cumsum(nonempty.astype(jnp.int32)) - 1, B)
    retval = jnp.zeros((B, H), jnp.float32).at[dest[perm]].set(h, mode="drop")
    return retval


# speedup vs baseline: 1.0046x; 1.0046x over previous
import functools

import jax
import jax.numpy as jnp
from jax.experimental import pallas as pl
from jax.experimental.pallas import tpu as pltpu
from jax.experimental.pallas import tpu_sc as plsc

_BLK = 1024
_CH = 8
_GW = 512
_EP = 128


def _sc_gather(emb_p, idx):
    n = idx.shape[0]
    ep = emb_p.shape[1]
    NC, NS = 2, 16
    NW = NC * NS
    per_w = n // NW
    SUP = 12800
    n_sup = per_w // SUP
    CHUNK = _GW
    n_ch = SUP // CHUNK
    mesh = plsc.VectorSubcoreMesh(core_axis_name="core", subcore_axis_name="subcore")

    @functools.partial(
        pl.kernel,
        out_type=jax.ShapeDtypeStruct((n, ep), emb_p.dtype),
        mesh=mesh,
        scratch_types=[
            pltpu.VMEM((SUP,), jnp.int32),
            pltpu.VMEM((CHUNK, ep), emb_p.dtype),
        ],
    )
    def k(emb_hbm, idx_hbm, out_hbm, idx_v, rows_v):
        core = jax.lax.axis_index("core")
        sub = jax.lax.axis_index("subcore")
        base = (sub * NC + core) * per_w

        @pl.loop(0, n_sup)
        def _sup(s):
            pltpu.sync_copy(idx_hbm.at[pl.ds(base + s * SUP, SUP)], idx_v)

            @pl.loop(0, n_ch)
            def _ch(c):
                pltpu.sync_copy(emb_hbm.at[idx_v.at[pl.ds(c * CHUNK, CHUNK)]],
                                rows_v)
                pltpu.sync_copy(
                    rows_v,
                    out_hbm.at[pl.ds(base + s * SUP + c * CHUNK, CHUNK)])

    return k(emb_p, idx)


def _gru_pallas(xe3, ls_col, lastchunk, wihT, whhT, b2):
    Bs, L, E = xe3.shape
    H = whhT.shape[0]
    G = whhT.shape[1]
    R = Bs // _BLK
    NT = L // _CH

    def body(s_ref, xe_ref, len_ref, wih_ref, whh_ref, b_ref, o_ref, h_ref):
        r = pl.program_id(0)
        c = pl.program_id(1)

        @pl.when(c == 0)
        def _init():
            h_ref[...] = jnp.zeros_like(h_ref)

        @pl.when(c <= s_ref[r])
        def _compute():
            h = h_ref[...]
            lens = len_ref[...]
            bih = b_ref[0:1, :]
            bhh = b_ref[1:2, :]
            for tt in range(_CH):
                t = c * _CH + tt
                xe_t = xe_ref[:, tt, :]
                gi = jnp.dot(xe_t, wih_ref[...],
                             preferred_element_type=jnp.float32) + bih
                gh = jnp.dot(h, whh_ref[...],
                             preferred_element_type=jnp.float32) + bhh
                gsum = gi + gh
                rz = jax.nn.sigmoid(gsum[:, : 2 * H])
                rr = rz[:, :H]
                zz = rz[:, H:]
                n = jnp.tanh(gi[:, 2 * H:] + rr * gh[:, 2 * H:])
                h_new = (1.0 - zz) * n + zz * h
                h = jnp.where(lens > t, h_new, h)
            h_ref[...] = h

        o_ref[...] = h_ref[...]

    return pl.pallas_call(
        body,
        grid_spec=pltpu.PrefetchScalarGridSpec(
            num_scalar_prefetch=1,
            grid=(R, NT),
            in_specs=[
                pl.BlockSpec((_BLK, _CH, E),
                             lambda r, c, s: (r, jnp.minimum(c, s[r]), 0)),
                pl.BlockSpec((_BLK, 1), lambda r, c, s: (r, 0)),
                pl.BlockSpec((E, G), lambda r, c, s: (0, 0)),
                pl.BlockSpec((H, G), lambda r, c, s: (0, 0)),
                pl.BlockSpec((8, G), lambda r, c, s: (0, 0)),
            ],
            out_specs=pl.BlockSpec((_BLK, H), lambda r, c, s: (r, 0)),
            scratch_shapes=[pltpu.VMEM((_BLK, H), jnp.float32)],
        ),
        out_shape=jax.ShapeDtypeStruct((Bs, H), jnp.float32),
        compiler_params=pltpu.CompilerParams(
            dimension_semantics=("parallel", "arbitrary")),
    )(lastchunk, xe3, ls_col, wihT, whhT, b2)


def kernel(x, emb, W_ih, W_hh, b_ih, b_hh):
    B, L = x.shape
    V, E = emb.shape
    H = W_hh.shape[1]

    l = jnp.sum(x != 0, axis=1).astype(jnp.int32)
    perm = jnp.argsort(-l)
    ls = l[perm]
    xs = x[perm]

    emb_p = jnp.pad(emb, ((0, 0), (0, _EP - E)))
    xe = _sc_gather(emb_p, xs.reshape(B * L))
    xe3 = xe.reshape(B, L, _EP)

    block_max = ls.reshape(B // _BLK, _BLK).max(axis=1)
    lastchunk = (jnp.maximum((block_max + _CH - 1) // _CH, 1) - 1).astype(jnp.int32)
    ls_col = ls[:, None]
    b2 = jnp.zeros((8, 3 * H), jnp.float32).at[0].set(b_ih).at[1].set(b_hh)

    wihT_p = jnp.pad(W_ih.T, ((0, _EP - E), (0, 0)))
    h = _gru_pallas(xe3, ls_col, lastchunk, wihT_p, W_hh.T, b2)

    nonempty = l != 0
    dest = jnp.where(nonempty, jnp.cumsum(nonempty.astype(jnp.int32)) - 1, B)
    retval = jnp.zeros((B, H), jnp.float32).at[dest[perm]].set(h, mode="drop")
    return retval


# speedup vs baseline: 22.7797x; 22.6764x over previous
import functools

import jax
import jax.numpy as jnp
from jax.experimental import pallas as pl
from jax.experimental.pallas import tpu as pltpu
from jax.experimental.pallas import tpu_sc as plsc

_BLK = 1024
_CH = 8
_GW = 512
_EP = 128
_NPAD = 4096


def _sc_gather(emb_p, idx):
    n = idx.shape[0]
    ep = emb_p.shape[1]
    NC, NS = 2, 16
    NW = NC * NS
    per_w = n // NW
    SUP = 12800
    n_sup = per_w // SUP
    CHUNK = _GW
    n_ch = SUP // CHUNK
    mesh = plsc.VectorSubcoreMesh(core_axis_name="core", subcore_axis_name="subcore")

    @functools.partial(
        pl.kernel,
        out_type=jax.ShapeDtypeStruct((n, ep), emb_p.dtype),
        mesh=mesh,
        scratch_types=[
            pltpu.VMEM((SUP,), jnp.int32),
            pltpu.VMEM((CHUNK, ep), emb_p.dtype),
        ],
    )
    def k(emb_hbm, idx_hbm, out_hbm, idx_v, rows_v):
        core = jax.lax.axis_index("core")
        sub = jax.lax.axis_index("subcore")
        base = (sub * NC + core) * per_w

        @pl.loop(0, n_sup)
        def _sup(s):
            pltpu.sync_copy(idx_hbm.at[pl.ds(base + s * SUP, SUP)], idx_v)

            @pl.loop(0, n_ch)
            def _ch(c):
                pltpu.sync_copy(emb_hbm.at[idx_v.at[pl.ds(c * CHUNK, CHUNK)]],
                                rows_v)
                pltpu.sync_copy(
                    rows_v,
                    out_hbm.at[pl.ds(base + s * SUP + c * CHUNK, CHUNK)])

    return k(emb_p, idx)


def _gru_pallas(xe3, ls_col, lastchunk, wihT, whhT, b2):
    Bs, L, E = xe3.shape
    H = whhT.shape[0]
    G = whhT.shape[1]
    R = Bs // _BLK
    NT = L // _CH

    def body(s_ref, xe_ref, len_ref, wih_ref, whh_ref, b_ref, o_ref, h_ref):
        r = pl.program_id(0)
        c = pl.program_id(1)

        @pl.when(c == 0)
        def _init():
            h_ref[...] = jnp.zeros_like(h_ref)

        @pl.when(c <= s_ref[r])
        def _compute():
            h = h_ref[...]
            lens = len_ref[...]
            bih = b_ref[0:1, :]
            bhh = b_ref[1:2, :]
            for tt in range(_CH):
                t = c * _CH + tt
                xe_t = xe_ref[:, tt, :]
                gi = jnp.dot(xe_t, wih_ref[...],
                             preferred_element_type=jnp.float32) + bih
                gh = jnp.dot(h, whh_ref[...],
                             preferred_element_type=jnp.float32) + bhh
                gsum = gi + gh
                rz = jax.nn.sigmoid(gsum[:, : 2 * H])
                rr = rz[:, :H]
                zz = rz[:, H:]
                n = jnp.tanh(gi[:, 2 * H:] + rr * gh[:, 2 * H:])
                h_new = (1.0 - zz) * n + zz * h
                h = jnp.where(lens > t, h_new, h)
            h_ref[...] = h

        o_ref[...] = h_ref[...]

    return pl.pallas_call(
        body,
        grid_spec=pltpu.PrefetchScalarGridSpec(
            num_scalar_prefetch=1,
            grid=(R, NT),
            in_specs=[
                pl.BlockSpec((_BLK, _CH, E),
                             lambda r, c, s: (r, jnp.minimum(c, s[r]), 0)),
                pl.BlockSpec((_BLK, 1), lambda r, c, s: (r, 0)),
                pl.BlockSpec((E, G), lambda r, c, s: (0, 0)),
                pl.BlockSpec((H, G), lambda r, c, s: (0, 0)),
                pl.BlockSpec((8, G), lambda r, c, s: (0, 0)),
            ],
            out_specs=pl.BlockSpec((_BLK, H), lambda r, c, s: (r, 0)),
            scratch_shapes=[pltpu.VMEM((_BLK, H), jnp.float32)],
        ),
        out_shape=jax.ShapeDtypeStruct((Bs, H), jnp.float32),
        compiler_params=pltpu.CompilerParams(
            dimension_semantics=("parallel", "arbitrary")),
    )(lastchunk, xe3, ls_col, wihT, whhT, b2)


def kernel(x, emb, W_ih, W_hh, b_ih, b_hh):
    B, L = x.shape
    V, E = emb.shape
    H = W_hh.shape[1]

    l = jnp.sum(x != 0, axis=1).astype(jnp.int32)
    perm = jnp.argsort(-l)
    ls = l[perm]
    xs = x[perm]

    emb_p = jnp.pad(emb, ((0, _NPAD), (0, _EP - E)))
    flat = xs.reshape(B * L)
    spread = (jnp.arange(B * L, dtype=jnp.int32) % _NPAD) + V
    xe = _sc_gather(emb_p, jnp.where(flat == 0, spread, flat))
    xe3 = xe.reshape(B, L, _EP)

    block_max = ls.reshape(B // _BLK, _BLK).max(axis=1)
    lastchunk = (jnp.maximum((block_max + _CH - 1) // _CH, 1) - 1).astype(jnp.int32)
    ls_col = ls[:, None]
    b2 = jnp.zeros((8, 3 * H), jnp.float32).at[0].set(b_ih).at[1].set(b_hh)

    wihT_p = jnp.pad(W_ih.T, ((0, _EP - E), (0, 0)))
    h = _gru_pallas(xe3, ls_col, lastchunk, wihT_p, W_hh.T, b2)

    nonempty = l != 0
    dest = jnp.where(nonempty, jnp.cumsum(nonempty.astype(jnp.int32)) - 1, B)
    retval = jnp.zeros((B, H), jnp.float32).at[dest[perm]].set(h, mode="drop")
    return retval
